# Initial kernel scaffold; baseline (speedup 1.0000x reference)
import jax
import jax.numpy as jnp
from jax.experimental import pallas as pl


def kernel(x, W, gamma, beta, edge_index, kernel_offsets):
    raise NotImplementedError("write your pallas kernel here")


# speedup vs baseline: 2.3535x; 2.3535x over previous
import jax
import jax.numpy as jnp
from jax import lax
from jax.experimental import pallas as pl
from jax.experimental.pallas import tpu as pltpu
from jax.experimental.pallas import tpu_sc as plsc

N = 10000
E = 320000
C_IN = 128
C_OUT = 128
K_VOL = 27
EPS = 1e-5

_CH = 128
_NW = 32
_EPT = -(-E // (_NW * _CH)) * _CH
_E_PAD = _EPT * _NW
_NCH = _EPT // _CH
_ACC_ROWS = 10240
_ROWS_PER_TILE = _ACC_ROWS // 16


def _xw_body(x_ref, w_ref, o_ref):
    o_ref[0] = jnp.dot(x_ref[...], w_ref[0], preferred_element_type=jnp.float32)


def _bn_body(p_ref, g_ref, b_ref, o_ref):
    s = p_ref[0, :N] + p_ref[1, :N]
    mean = jnp.mean(s, axis=0, keepdims=True)
    d = s - mean
    var = jnp.mean(d * d, axis=0, keepdims=True)
    o_ref[...] = d / jnp.sqrt(var + EPS) * g_ref[...] + b_ref[...]


def _sc_body(xw_hbm, src_hbm, dst_hbm, off_hbm, zeros_hbm, out_hbm,
             src_v, dst_v, off_v, idx_v, dstb_v, rows_v, acc_sh, sem):
    cid = lax.axis_index("c")
    sid = lax.axis_index("s")
    wid = sid * 2 + cid

    r0 = sid * _ROWS_PER_TILE
    pltpu.sync_copy(zeros_hbm.at[pl.ds(r0, _ROWS_PER_TILE)],
                    acc_sh.at[pl.ds(r0, _ROWS_PER_TILE)])
    plsc.subcore_barrier()

    base = wid * _EPT
    pltpu.sync_copy(src_hbm.at[pl.ds(base, _EPT)], src_v)
    pltpu.sync_copy(dst_hbm.at[pl.ds(base, _EPT)], dst_v)
    pltpu.sync_copy(off_hbm.at[pl.ds(base, _EPT)], off_v)

    def chunk(i, carry):
        for j in range(_CH // 16):
            sl_in = pl.ds(i * _CH + j * 16, 16)
            sl_out = pl.ds(j * 16, 16)
            idx_v[sl_out] = off_v[sl_in] * N + src_v[sl_in]
            dstb_v[sl_out] = dst_v[sl_in]
        pltpu.async_copy(xw_hbm.at[idx_v], rows_v, sem).wait()
        pltpu.sync_copy(rows_v, acc_sh.at[dstb_v], add=True)
        return carry

    lax.fori_loop(0, _NCH, chunk, 0)

    plsc.subcore_barrier()
    pltpu.sync_copy(acc_sh.at[pl.ds(r0, _ROWS_PER_TILE)],
                    out_hbm.at[cid, pl.ds(r0, _ROWS_PER_TILE)])


def kernel(x, W, gamma, beta, edge_index, kernel_offsets):
    src = edge_index[0]
    dst = edge_index[1]
    pad = _E_PAD - E
    src_p = jnp.concatenate([src, jnp.zeros((pad,), jnp.int32)])
    dst_p = jnp.concatenate([dst, jnp.full((pad,), N, jnp.int32)])
    off_p = jnp.concatenate([kernel_offsets, jnp.zeros((pad,), jnp.int32)])
    zeros = jnp.zeros((_ACC_ROWS, C_OUT), jnp.float32)

    block_n = 2000
    xw = pl.pallas_call(
        _xw_body,
        grid=(N // block_n, K_VOL),
        in_specs=[pl.BlockSpec((block_n, C_IN), lambda nb, k: (nb, 0)),
                  pl.BlockSpec((1, C_IN, C_OUT), lambda nb, k: (k, 0, 0))],
        out_specs=pl.BlockSpec((1, block_n, C_OUT), lambda nb, k: (k, nb, 0)),
        out_shape=jax.ShapeDtypeStruct((K_VOL, N, C_OUT), jnp.float32),
    )(x, W)
    xw_flat = xw.reshape(K_VOL * N, C_OUT)

    mesh = plsc.VectorSubcoreMesh(core_axis_name="c", subcore_axis_name="s")
    partial = pl.kernel(
        _sc_body,
        out_type=jax.ShapeDtypeStruct((2, _ACC_ROWS, C_OUT), jnp.float32),
        mesh=mesh,
        scratch_types=[
            pltpu.VMEM((_EPT,), jnp.int32),
            pltpu.VMEM((_EPT,), jnp.int32),
            pltpu.VMEM((_EPT,), jnp.int32),
            pltpu.VMEM((_CH,), jnp.int32),
            pltpu.VMEM((_CH,), jnp.int32),
            pltpu.VMEM((_CH, C_OUT), jnp.float32),
            pltpu.VMEM_SHARED((_ACC_ROWS, C_OUT), jnp.float32),
            pltpu.SemaphoreType.DMA,
        ],
    )(xw_flat, src_p, dst_p, off_p, zeros)

    return pl.pallas_call(
        _bn_body,
        out_shape=jax.ShapeDtypeStruct((N, C_OUT), jnp.float32),
    )(partial, gamma.reshape(1, C_OUT), beta.reshape(1, C_OUT))
